# baseline (speedup 1.0000x reference)
import functools
import math

import jax
import jax.numpy as jnp
from jax import lax
from jax.experimental import pallas as pl
from jax.experimental.pallas import tpu as pltpu
from jax.experimental.pallas import tpu_sc as plsc

NC = 2
NS = 16
NW = NC * NS
LANES = 16
CHUNK = 128

_PARAMS = pltpu.CompilerParams(
    use_tc_tiling_on_sc=True, needs_layout_passes=False)

_MESH = plsc.VectorSubcoreMesh(
    core_axis_name="c", subcore_axis_name="s",
    num_cores=NC, num_subcores=NS)


def _repack_table(wt, tailx, *, scale):
    D, V = wt.shape
    VB = V // CHUNK
    TAIL = V - VB * CHUNK
    assert TAIL % 2 == 0
    PER_W = -(-VB // NW)
    GROUPS = -(-PER_W // 2)

    @functools.partial(
        pl.kernel,
        out_type=jax.ShapeDtypeStruct((V // 2, 2 * D), jnp.float32),
        mesh=_MESH,
        scratch_types=[
            *([pltpu.VMEM((D, CHUNK), jnp.float32)] * 2),
            *([pltpu.VMEM((CHUNK // 2, 2 * D), jnp.float32)] * 2),
            *([pltpu.SemaphoreType.DMA] * 4),
        ],
        compiler_params=_PARAMS,
    )
    def k(wt_hbm, tailx_hbm, x_hbm, blk0, blk1, ob0, ob1, gs0, gs1, ss0, ss1):
        blks, obs = (blk0, blk1), (ob0, ob1)
        gsem, ssem = (gs0, gs1), (ss0, ss1)
        wid = lax.axis_index("s") * NC + lax.axis_index("c")
        lane = jax.lax.iota(jnp.int32, LANES)
        rowv = jax.lax.shift_right_logical(lane, 1)
        colb = (lane & 1) * D
        nst = lax.div(VB - wid + NW - 1, NW)

        def start_load(b, j):
            pltpu.async_copy(
                wt_hbm.at[:, pl.ds(j * CHUNK, CHUNK)], blks[b], gsem[b])

        def transpose_block(b, gmax):
            @pl.loop(0, D)
            def _d(d):
                for g in range(gmax):
                    vals = blks[b][d, pl.ds(g * LANES, LANES)] * scale
                    plsc.store_scatter(
                        obs[b], [g * (LANES // 2) + rowv, colb + d], vals)

        for b in range(2):
            @pl.when(b < nst)
            def _():
                start_load(b, wid + b * NW)

        @pl.loop(0, GROUPS)
        def _grp(i0):
            for b in range(2):
                i = i0 * 2 + b
                j = wid + i * NW

                @pl.when(i < nst)
                def _():
                    pltpu.make_async_copy(
                        wt_hbm.at[:, pl.ds(j * CHUNK, CHUNK)],
                        blks[b], gsem[b]).wait()

                    @pl.when(i >= 2)
                    def _():
                        prev = x_hbm.at[pl.ds((j - 2 * NW) * (CHUNK // 2),
                                              CHUNK // 2)]
                        pltpu.make_async_copy(obs[b], prev, ssem[b]).wait()

                    transpose_block(b, CHUNK // LANES)
                    pltpu.async_copy(
                        obs[b],
                        x_hbm.at[pl.ds(j * (CHUNK // 2), CHUNK // 2)],
                        ssem[b])

                    @pl.when(i + 2 < nst)
                    def _():
                        start_load(b, j + 2 * NW)

        for b in range(2):
            @pl.when(nst >= 2 - b)
            def _():
                i_last = ((nst - 1 - b) // 2) * 2 + b
                j_last = wid + i_last * NW
                pltpu.make_async_copy(
                    obs[b],
                    x_hbm.at[pl.ds(j_last * (CHUNK // 2), CHUNK // 2)],
                    ssem[b]).wait()

        if TAIL:
            @pl.when(wid == 0)
            def _():
                pltpu.sync_copy(tailx_hbm, ob0.at[pl.ds(0, TAIL // 2)])
                pltpu.sync_copy(
                    ob0.at[pl.ds(0, TAIL // 2)],
                    x_hbm.at[pl.ds(VB * (CHUNK // 2), TAIL // 2)])

    return k(wt, tailx)


def _gather_tokens(tokens_3d, x, *, nchunk):
    V2, D2 = x.shape
    D = D2 // 2
    B = NW * nchunk * CHUNK
    NBUF = 2

    @functools.partial(
        pl.kernel,
        out_type=jax.ShapeDtypeStruct((B, D), jnp.float32),
        mesh=_MESH,
        scratch_types=[
            pltpu.VMEM((nchunk, CHUNK), jnp.int32),
            *([pltpu.VMEM((CHUNK,), jnp.int32)] * NBUF),
            *([pltpu.VMEM((CHUNK, D2), jnp.float32)] * NBUF),
            *([pltpu.VMEM((CHUNK, D), jnp.float32)] * NBUF),
            *([pltpu.SemaphoreType.DMA] * (2 * NBUF)),
        ],
        compiler_params=_PARAMS,
    )
    def k(tokens_hbm, x_hbm, out_hbm, idx_v, *refs):
        ridx = refs[:NBUF]
        rows = refs[NBUF:2 * NBUF]
        outv = refs[2 * NBUF:3 * NBUF]
        gsem = refs[3 * NBUF:4 * NBUF]
        ssem = refs[4 * NBUF:]
        wid = lax.axis_index("s") * NC + lax.axis_index("c")
        base = wid * (nchunk * CHUNK)
        lane = jax.lax.iota(jnp.int32, LANES)
        pltpu.sync_copy(tokens_hbm.at[wid], idx_v)

        def start_gather(b, j):
            @pl.loop(0, CHUNK // LANES)
            def _(g):
                sl = pl.ds(g * LANES, LANES)
                ridx[b][sl] = jax.lax.shift_right_logical(idx_v[j, sl], 1)
            pltpu.async_copy(x_hbm.at[ridx[b]], rows[b], gsem[b])

        for b in range(NBUF):
            start_gather(b, b)

        @pl.loop(0, nchunk, step=NBUF)
        def _grp(j0):
            for b in range(NBUF):
                j = j0 + b
                pltpu.make_async_copy(
                    x_hbm.at[ridx[b]], rows[b], gsem[b]).wait()

                @pl.when(j >= NBUF)
                def _():
                    prev = out_hbm.at[pl.ds(base + (j - NBUF) * CHUNK, CHUNK)]
                    pltpu.make_async_copy(outv[b], prev, ssem[b]).wait()

                @pl.loop(0, CHUNK // LANES)
                def _g(g):
                    trow = g * LANES + lane
                    half = (idx_v[j, pl.ds(g * LANES, LANES)] & 1) * D
                    for c in range(D):
                        vals = plsc.load_gather(rows[b], [trow, half + c])
                        plsc.store_scatter(
                            outv[b],
                            [trow, jnp.full((LANES,), c, jnp.int32)], vals)

                pltpu.async_copy(
                    outv[b],
                    out_hbm.at[pl.ds(base + j * CHUNK, CHUNK)], ssem[b])

                @pl.when(j + NBUF < nchunk)
                def _():
                    start_gather(b, j + NBUF)

        for b in range(NBUF):
            j = nchunk - NBUF + b
            pltpu.make_async_copy(
                outv[b],
                out_hbm.at[pl.ds(base + j * CHUNK, CHUNK)], ssem[b]).wait()

    return k(tokens_3d, x)


def kernel(tokens, embedding_weight):
    B0, S = tokens.shape
    V, D = embedding_weight.shape
    B = B0 * S
    assert B % (NW * CHUNK) == 0 and V % 2 == 0 and 2 * D == 128
    nchunk = B // (NW * CHUNK)
    scale = math.sqrt(D)
    vb = (V // CHUNK) * CHUNK
    tailx = jnp.concatenate(
        [embedding_weight[vb::2] * scale,
         embedding_weight[vb + 1::2] * scale], axis=1)
    x = _repack_table(embedding_weight.T, tailx, scale=scale)
    flat = tokens.reshape(NW, nchunk, CHUNK).astype(jnp.int32)
    out = _gather_tokens(flat, x, nchunk=nchunk)
    return out.reshape(B0, S, D)


# speedup vs baseline: 2.6570x; 2.6570x over previous
import functools
import math

import jax
import jax.numpy as jnp
from jax import lax
from jax.experimental import pallas as pl
from jax.experimental.pallas import tpu as pltpu
from jax.experimental.pallas import tpu_sc as plsc

NC = 2
NS = 16
NW = NC * NS
LANES = 16
CHUNK = 128

_PARAMS = pltpu.CompilerParams(
    use_tc_tiling_on_sc=True, needs_layout_passes=False)

_MESH = plsc.VectorSubcoreMesh(
    core_axis_name="c", subcore_axis_name="s",
    num_cores=NC, num_subcores=NS)


def _repack_table(wt, tailx, *, scale):
    D, V = wt.shape
    VB = V // CHUNK
    TAIL = V - VB * CHUNK
    PER_W = -(-VB // NW)
    GROUPS = -(-PER_W // 2)

    @functools.partial(
        pl.kernel,
        out_type=jax.ShapeDtypeStruct((V // 2, 2 * D), jnp.float32),
        mesh=_MESH,
        scratch_types=[
            *([pltpu.VMEM((D, CHUNK), jnp.float32)] * 2),
            *([pltpu.VMEM((CHUNK // 2, 2 * D), jnp.float32)] * 2),
            *([pltpu.SemaphoreType.DMA] * 4),
        ],
        compiler_params=_PARAMS,
    )
    def k(wt_hbm, tailx_hbm, x_hbm, blk0, blk1, ob0, ob1, gs0, gs1, ss0, ss1):
        blks, obs = (blk0, blk1), (ob0, ob1)
        gsem, ssem = (gs0, gs1), (ss0, ss1)
        wid = lax.axis_index("s") * NC + lax.axis_index("c")
        lane = jax.lax.iota(jnp.int32, LANES)
        nst = lax.div(VB - wid + NW - 1, NW)

        def start_load(b, j):
            pltpu.async_copy(
                wt_hbm.at[:, pl.ds(j * CHUNK, CHUNK)], blks[b], gsem[b])

        def transpose_block(b):
            for v0 in range(0, CHUNK, LANES):
                vv = v0 + lane
                rowv = jax.lax.shift_right_logical(vv, 1)
                colb = (vv & 1) * D

                @pl.loop(0, LANES)
                def _kk(kk):
                    perm = (lane + kk) & (LANES - 1)
                    for dd in range(D // LANES):
                        d = dd * LANES + perm
                        vals = plsc.load_gather(blks[b], [d, vv]) * scale
                        plsc.store_scatter(obs[b], [rowv, colb + d], vals)

        for b in range(2):
            start_load(b, wid + b * NW)

        @pl.loop(0, GROUPS)
        def _grp(i0):
            for b in range(2):
                i = i0 * 2 + b
                j = wid + i * NW

                @pl.when(i < nst)
                def _():
                    pltpu.make_async_copy(
                        wt_hbm.at[:, pl.ds(j * CHUNK, CHUNK)],
                        blks[b], gsem[b]).wait()

                    @pl.when(i >= 2)
                    def _():
                        prev = x_hbm.at[pl.ds((j - 2 * NW) * (CHUNK // 2),
                                              CHUNK // 2)]
                        pltpu.make_async_copy(obs[b], prev, ssem[b]).wait()

                    transpose_block(b)
                    pltpu.async_copy(
                        obs[b],
                        x_hbm.at[pl.ds(j * (CHUNK // 2), CHUNK // 2)],
                        ssem[b])

                    @pl.when(i + 2 < nst)
                    def _():
                        start_load(b, j + 2 * NW)

        for b in range(2):
            i_last = ((nst - 1 - b) // 2) * 2 + b
            j_last = wid + i_last * NW
            pltpu.make_async_copy(
                obs[b],
                x_hbm.at[pl.ds(j_last * (CHUNK // 2), CHUNK // 2)],
                ssem[b]).wait()

        if TAIL:
            @pl.when(wid == 0)
            def _():
                pltpu.sync_copy(tailx_hbm, ob0.at[pl.ds(0, TAIL // 2)])
                pltpu.sync_copy(
                    ob0.at[pl.ds(0, TAIL // 2)],
                    x_hbm.at[pl.ds(VB * (CHUNK // 2), TAIL // 2)])

    return k(wt, tailx)


def _gather_tokens(tokens_t, x):
    V2, D2 = x.shape
    D = D2 // 2
    S, B = tokens_t.shape
    NBUF = 2

    @functools.partial(
        pl.kernel,
        out_type=jax.ShapeDtypeStruct((S, D, B), jnp.float32),
        mesh=_MESH,
        scratch_types=[
            pltpu.VMEM((S, CHUNK), jnp.int32),
            *([pltpu.VMEM((CHUNK,), jnp.int32)] * NBUF),
            *([pltpu.VMEM((CHUNK, D2), jnp.float32)] * NBUF),
            *([pltpu.VMEM((D, CHUNK), jnp.float32)] * NBUF),
            *([pltpu.SemaphoreType.DMA] * (2 * NBUF)),
        ],
        compiler_params=_PARAMS,
    )
    def k(tok_hbm, x_hbm, out_hbm, idx_v, *refs):
        ridx = refs[:NBUF]
        rows = refs[NBUF:2 * NBUF]
        outt = refs[2 * NBUF:3 * NBUF]
        gsem = refs[3 * NBUF:4 * NBUF]
        ssem = refs[4 * NBUF:]
        wid = lax.axis_index("s") * NC + lax.axis_index("c")
        b0 = wid * CHUNK
        lane = jax.lax.iota(jnp.int32, LANES)
        pltpu.sync_copy(tok_hbm.at[:, pl.ds(b0, CHUNK)], idx_v)

        def start_gather(b, s):
            @pl.loop(0, CHUNK // LANES)
            def _(g):
                sl = pl.ds(g * LANES, LANES)
                ridx[b][sl] = jax.lax.shift_right_logical(idx_v[s, sl], 1)
            pltpu.async_copy(x_hbm.at[ridx[b]], rows[b], gsem[b])

        def select_transpose(b, s):
            for i0 in range(0, CHUNK, LANES):
                @pl.loop(0, LANES)
                def _kk(kk):
                    perm = (lane + kk) & (LANES - 1)
                    iv = i0 + perm
                    tv = plsc.load_gather(idx_v.at[s], [iv])
                    half = (tv & 1) * D
                    for dd in range(D // LANES):
                        d = dd * LANES + lane
                        vals = plsc.load_gather(rows[b], [iv, half + d])
                        plsc.store_scatter(outt[b], [d, iv], vals)

        for b in range(NBUF):
            start_gather(b, b)

        @pl.loop(0, S, step=NBUF)
        def _grp(s0):
            for b in range(NBUF):
                s = s0 + b
                pltpu.make_async_copy(
                    x_hbm.at[ridx[b]], rows[b], gsem[b]).wait()

                @pl.when(s >= NBUF)
                def _():
                    prev = out_hbm.at[s - NBUF, :, pl.ds(b0, CHUNK)]
                    pltpu.make_async_copy(outt[b], prev, ssem[b]).wait()

                select_transpose(b, s)
                pltpu.async_copy(
                    outt[b], out_hbm.at[s, :, pl.ds(b0, CHUNK)], ssem[b])

                @pl.when(s + NBUF < S)
                def _():
                    start_gather(b, s + NBUF)

        for b in range(NBUF):
            s = S - NBUF + b
            pltpu.make_async_copy(
                outt[b], out_hbm.at[s, :, pl.ds(b0, CHUNK)], ssem[b]).wait()

    return k(tokens_t, x)


def kernel(tokens, embedding_weight):
    B0, S = tokens.shape
    V, D = embedding_weight.shape
    assert B0 == NW * CHUNK and V % 2 == 0 and 2 * D == 128 and S % 2 == 0
    scale = math.sqrt(D)
    vb = (V // CHUNK) * CHUNK
    tailx = jnp.concatenate(
        [embedding_weight[vb::2] * scale,
         embedding_weight[vb + 1::2] * scale], axis=1)
    x = _repack_table(embedding_weight.T, tailx, scale=scale)
    out_sdb = _gather_tokens(tokens.T.astype(jnp.int32), x)
    return out_sdb.transpose(2, 0, 1)
